# Initial kernel scaffold; baseline (speedup 1.0000x reference)
import jax
import jax.numpy as jnp
from jax.experimental import pallas as pl


def kernel(x, edge_index, edge_weight, W1, b1, W2, b2, Wfc, bfc):
    raise NotImplementedError("write your pallas kernel here")


# speedup vs baseline: 8.1998x; 8.1998x over previous
import functools

import jax
import jax.numpy as jnp
from jax import lax
from jax.experimental import pallas as pl
from jax.experimental.pallas import tpu as pltpu
from jax.experimental.pallas import tpu_sc as plsc

N = 100000
E = 1600000
NCLS = 7
HID = 128

NC = 2
NS = 16
L = 16

NP = 100096
NR = NP // 2
NDUMP = 1024
ACC = NR + NDUMP
CH = 10000
NCHUNK = E // NS // CH
NVEC = CH // L
NBLK = 78
PBLK = NBLK + 1

_mesh = plsc.VectorSubcoreMesh(
    core_axis_name="c", subcore_axis_name="s", num_cores=NC, num_subcores=NS)

_PER_TEC = ACC // NS
_OUT_TEC = NR // NS


def _zero_rows(rows, width):
  z = jnp.zeros((L,), jnp.float32)

  def body(i, _):
    for j in range(width // L):
      rows[i, pl.ds(j * L, L)] = z
    return 0

  lax.fori_loop(0, 128, body, 0)


@functools.partial(
    pl.kernel,
    out_type=(
        jax.ShapeDtypeStruct((NC * NR,), jnp.float32),
        jax.ShapeDtypeStruct((NC * NS * NCHUNK, PBLK, 128), jnp.int32),
    ),
    mesh=_mesh,
    scratch_types=[
        pltpu.VMEM((CH,), jnp.int32),
        pltpu.VMEM((CH,), jnp.float32),
        pltpu.VMEM((PBLK, 128), jnp.int32),
        pltpu.VMEM((3192,), jnp.float32),
        pltpu.VMEM_SHARED((ACC,), jnp.float32),
    ],
    compiler_params=pltpu.CompilerParams(use_tc_tiling_on_sc=False),
)
def _k1_deg(dst_hbm, w_hbm, out_hbm, dlocb_hbm, dst_v, w_v, pos2d, zbuf,
            deg_sp):
  c = lax.axis_index("c")
  s = lax.axis_index("s")
  half_lo = c * NR

  z = jnp.zeros((L,), jnp.float32)

  def zb(i, _):
    zbuf[pl.ds(i * L, L)] = z
    return 0

  lax.fori_loop(0, 3192 // L, zb, 0)
  pltpu.sync_copy(zbuf, deg_sp.at[pl.ds(s * _PER_TEC, _PER_TEC)])
  plsc.subcore_barrier()

  @pl.loop(0, NCHUNK)
  def chunk_loop(ci):
    base = s * (E // NS) + ci * CH
    pltpu.sync_copy(dst_hbm.at[pl.ds(base, CH)], dst_v)
    pltpu.sync_copy(w_hbm.at[pl.ds(base, CH)], w_v)

    def vec(i, _):
      d = dst_v[pl.ds(i * L, L)]
      dl = d - half_lo
      in_rng = (dl >= 0) & (dl < NR)
      pos2d[i >> 3, pl.ds((i & 7) * L, L)] = jnp.where(
          in_rng, dl, NR + (d & (NDUMP - 1)))
      return 0

    lax.fori_loop(0, NVEC, vec, 0)
    pltpu.sync_copy(pos2d, dlocb_hbm.at[(c * NS + s) * NCHUNK + ci])

    def blk(jb, _):
      pltpu.sync_copy(w_v.at[pl.ds(jb * 128, 128)],
                      deg_sp.at[pos2d.at[jb]], add=True)
      return 0

    lax.fori_loop(0, NBLK, blk, 0)
    pltpu.sync_copy(w_v.at[pl.ds(NBLK * 128, 16)],
                    deg_sp.at[pos2d.at[NBLK, pl.ds(0, 16)]], add=True)

  plsc.subcore_barrier()
  pltpu.sync_copy(deg_sp.at[pl.ds(s * _OUT_TEC, _OUT_TEC)],
                  zbuf.at[pl.ds(0, _OUT_TEC)])
  pltpu.sync_copy(zbuf.at[pl.ds(0, _OUT_TEC)],
                  out_hbm.at[pl.ds(c * NR + s * _OUT_TEC, _OUT_TEC)])


def _agg_body(dlocb_hbm, src_hbm, w_hbm, g_hbm, out_hbm, src_v, w_v, pos2d,
              rows0, rows1, acc_sp, sem0, sem1):
  c = lax.axis_index("c")
  s = lax.axis_index("s")

  _zero_rows(rows0, 16)
  for j in range(_PER_TEC // 128):
    pltpu.sync_copy(rows0, acc_sp.at[pl.ds(s * _PER_TEC + j * 128, 128)])
  rem = _PER_TEC % 128
  if rem:
    pltpu.sync_copy(rows0.at[pl.ds(0, rem)],
                    acc_sp.at[pl.ds(s * _PER_TEC + (_PER_TEC // 128) * 128,
                                    rem)])
  plsc.subcore_barrier()

  def scale_scatter(rows, bb, jb):
    def scale(g, _):
      wvec = w_v[pl.ds(bb + g * L, L)]
      for i in range(L):
        k = g * L + i
        rows[k, pl.ds(0, L)] = rows[k, pl.ds(0, L)] * wvec[i]
      return 0

    lax.fori_loop(0, 128 // L, scale, 0)
    pltpu.sync_copy(rows, acc_sp.at[pos2d.at[jb]], add=True)

  @pl.loop(0, NCHUNK)
  def chunk_loop(ci):
    base = s * (E // NS) + ci * CH
    pltpu.sync_copy(dlocb_hbm.at[(c * NS + s) * NCHUNK + ci], pos2d)
    pltpu.sync_copy(src_hbm.at[pl.ds(base, CH)], src_v)
    pltpu.sync_copy(w_hbm.at[pl.ds(base, CH)], w_v)

    pltpu.async_copy(g_hbm.at[src_v.at[pl.ds(0, 128)]], rows0, sem0)

    @pl.loop(0, NBLK, step=2)
    def blk(jb):
      bb = jb * 128
      pltpu.async_copy(g_hbm.at[src_v.at[pl.ds(bb + 128, 128)]], rows1, sem1)
      pltpu.make_async_copy(g_hbm.at[src_v.at[pl.ds(bb, 128)]], rows0,
                            sem0).wait()
      scale_scatter(rows0, bb, jb)

      @pl.when(jb + 2 < NBLK)
      def _():
        pltpu.async_copy(g_hbm.at[src_v.at[pl.ds(bb + 256, 128)]], rows0,
                         sem0)

      pltpu.make_async_copy(g_hbm.at[src_v.at[pl.ds(bb + 128, 128)]], rows1,
                            sem1).wait()
      scale_scatter(rows1, bb + 128, jb + 1)

    bb = NBLK * 128
    pltpu.async_copy(g_hbm.at[src_v.at[pl.ds(bb, 16)]],
                     rows0.at[pl.ds(0, 16)], sem0).wait()
    wvec = w_v[pl.ds(bb, L)]
    for i in range(L):
      rows0[i, pl.ds(0, L)] = rows0[i, pl.ds(0, L)] * wvec[i]
    pltpu.sync_copy(rows0.at[pl.ds(0, 16)],
                    acc_sp.at[pos2d.at[NBLK, pl.ds(0, 16)]], add=True)

  plsc.subcore_barrier()
  nfull = _OUT_TEC // 128
  for j in range(nfull):
    pltpu.sync_copy(acc_sp.at[pl.ds(s * _OUT_TEC + j * 128, 128)], rows0)
    pltpu.sync_copy(rows0,
                    out_hbm.at[pl.ds(c * NR + s * _OUT_TEC + j * 128, 128)])
  rem = _OUT_TEC - nfull * 128
  if rem:
    pltpu.sync_copy(acc_sp.at[pl.ds(s * _OUT_TEC + nfull * 128, rem)],
                    rows0.at[pl.ds(0, rem)])
    pltpu.sync_copy(rows0.at[pl.ds(0, rem)],
                    out_hbm.at[pl.ds(c * NR + s * _OUT_TEC + nfull * 128,
                                     rem)])


_agg16 = functools.partial(
    pl.kernel,
    out_type=jax.ShapeDtypeStruct((NP, 16), jnp.float32),
    mesh=_mesh,
    scratch_types=[
        pltpu.VMEM((CH,), jnp.int32),
        pltpu.VMEM((CH,), jnp.float32),
        pltpu.VMEM((PBLK, 128), jnp.int32),
        pltpu.VMEM((128, 16), jnp.float32),
        pltpu.VMEM((128, 16), jnp.float32),
        pltpu.VMEM_SHARED((ACC, 16), jnp.float32),
        pltpu.SemaphoreType.DMA,
        pltpu.SemaphoreType.DMA,
    ],
    compiler_params=pltpu.CompilerParams(use_tc_tiling_on_sc=False),
)(_agg_body)


def _t1_body(deg_ref, x_ref, dis_ref, gx_ref):
  deg = deg_ref[...] + 1.0
  dis = jnp.where(deg > 0, lax.rsqrt(deg), 0.0)
  dis_ref[...] = dis
  gx_ref[...] = x_ref[...] * dis


def _t2_body(aggx_ref, gx_ref, dis_ref, w1t_ref, b1_ref, g1_ref):
  dis = dis_ref[...]
  t = (aggx_ref[...] + gx_ref[...]) * dis
  h = jnp.dot(t, w1t_ref[...], preferred_element_type=jnp.float32)
  g1_ref[...] = jnp.maximum(h + b1_ref[...], 0.0) * dis


def _t3_body(agg1_ref, g1_ref, dis_ref, w2t_ref, b2_ref, wfct_ref, bfc_ref,
             o_ref):
  t = (agg1_ref[...] + g1_ref[...]) * dis_ref[...]
  h2 = jnp.maximum(
      jnp.dot(t, w2t_ref[...], preferred_element_type=jnp.float32)
      + b2_ref[...], 0.0)
  o_ref[...] = (jnp.dot(h2, wfct_ref[...], preferred_element_type=jnp.float32)
                + bfc_ref[...])


def kernel(x, edge_index, edge_weight, W1, b1, W2, b2, Wfc, bfc):
  src = edge_index[0]
  dst = edge_index[1]
  w = edge_weight

  xpad = jnp.pad(x, ((0, NP - N), (0, 16 - NCLS)))

  deg, dlocb = _k1_deg(dst, w)
  deg = deg[:, None]

  rb1 = 6256
  dis, gx = pl.pallas_call(
      _t1_body,
      grid=(NP // rb1,),
      in_specs=[
          pl.BlockSpec((rb1, 1), lambda i: (i, 0)),
          pl.BlockSpec((rb1, 16), lambda i: (i, 0)),
      ],
      out_specs=[
          pl.BlockSpec((rb1, 1), lambda i: (i, 0)),
          pl.BlockSpec((rb1, 16), lambda i: (i, 0)),
      ],
      out_shape=[
          jax.ShapeDtypeStruct((NP, 1), jnp.float32),
          jax.ShapeDtypeStruct((NP, 16), jnp.float32),
      ],
  )(deg, xpad)

  aggx = _agg16(dlocb, src, w, gx)

  w1t = jnp.pad(W1.T, ((0, 16 - NCLS), (0, 0)))
  rb2 = 3128
  g1 = pl.pallas_call(
      _t2_body,
      grid=(NP // rb2,),
      in_specs=[
          pl.BlockSpec((rb2, 16), lambda i: (i, 0)),
          pl.BlockSpec((rb2, 16), lambda i: (i, 0)),
          pl.BlockSpec((rb2, 1), lambda i: (i, 0)),
          pl.BlockSpec((16, HID), lambda i: (0, 0)),
          pl.BlockSpec((1, HID), lambda i: (0, 0)),
      ],
      out_specs=pl.BlockSpec((rb2, HID), lambda i: (i, 0)),
      out_shape=jax.ShapeDtypeStruct((NP, HID), jnp.float32),
  )(aggx, gx, dis, w1t, b1[None, :])

  agg1 = jnp.concatenate(
      [_agg16(dlocb, src, w,
              lax.slice(g1, (0, r * 16), (NP, (r + 1) * 16)))
       for r in range(8)], axis=1)

  rb3 = 3128
  out = pl.pallas_call(
      _t3_body,
      grid=(NP // rb3,),
      in_specs=[
          pl.BlockSpec((rb3, HID), lambda i: (i, 0)),
          pl.BlockSpec((rb3, HID), lambda i: (i, 0)),
          pl.BlockSpec((rb3, 1), lambda i: (i, 0)),
          pl.BlockSpec((HID, HID), lambda i: (0, 0)),
          pl.BlockSpec((1, HID), lambda i: (0, 0)),
          pl.BlockSpec((HID, NCLS), lambda i: (0, 0)),
          pl.BlockSpec((1, NCLS), lambda i: (0, 0)),
      ],
      out_specs=pl.BlockSpec((rb3, NCLS), lambda i: (i, 0)),
      out_shape=jax.ShapeDtypeStruct((NP, NCLS), jnp.float32),
  )(agg1, g1, dis, W2.T, b2[None, :], Wfc.T, bfc[None, :])

  return out[:N]


# speedup vs baseline: 11.8893x; 1.4500x over previous
import functools

import jax
import jax.numpy as jnp
from jax import lax
from jax.experimental import pallas as pl
from jax.experimental.pallas import tpu as pltpu
from jax.experimental.pallas import tpu_sc as plsc

N = 100000
E = 1600000
NCLS = 7
HID = 128

NC = 2
NS = 16
L = 16

NP = 100096
NR = NP // 2
NDUMP = 1024
ACC = NR + NDUMP
ET = E // NS
SEG = 2048
NSEG = ET // SEG
TAIL = ET - NSEG * SEG

_mesh = plsc.VectorSubcoreMesh(
    core_axis_name="c", subcore_axis_name="s", num_cores=NC, num_subcores=NS)

_PER_TEC = ACC // NS
_OUT_TEC = NR // NS


@functools.partial(
    pl.kernel,
    out_type=(
        jax.ShapeDtypeStruct((NC * NR,), jnp.float32),
        jax.ShapeDtypeStruct((NC * E,), jnp.int32),
    ),
    mesh=_mesh,
    scratch_types=[
        pltpu.VMEM((SEG,), jnp.int32),
        pltpu.VMEM((SEG,), jnp.float32),
        pltpu.VMEM((SEG,), jnp.int32),
        pltpu.VMEM((3192,), jnp.float32),
        pltpu.VMEM_SHARED((ACC,), jnp.float32),
    ],
    compiler_params=pltpu.CompilerParams(use_tc_tiling_on_sc=False),
)
def _k1_deg(dst_hbm, w_hbm, out_hbm, dlocb_hbm, dst_v, w_v, pos_v, zbuf,
            deg_sp):
  c = lax.axis_index("c")
  s = lax.axis_index("s")
  half_lo = c * NR

  z = jnp.zeros((L,), jnp.float32)

  def zb(i, _):
    zbuf[pl.ds(i * L, L)] = z
    return 0

  lax.fori_loop(0, 3192 // L, zb, 0)
  pltpu.sync_copy(zbuf, deg_sp.at[pl.ds(s * _PER_TEC, _PER_TEC)])
  plsc.subcore_barrier()

  def do_seg(base, nedge):
    pltpu.sync_copy(dst_hbm.at[pl.ds(base, nedge)], dst_v.at[pl.ds(0, nedge)])
    pltpu.sync_copy(w_hbm.at[pl.ds(base, nedge)], w_v.at[pl.ds(0, nedge)])

    def vec(i, _):
      d = dst_v[pl.ds(i * L, L)]
      dl = d - half_lo
      in_rng = (dl >= 0) & (dl < NR)
      pos_v[pl.ds(i * L, L)] = jnp.where(in_rng, dl, NR + (d & (NDUMP - 1)))
      return 0

    lax.fori_loop(0, nedge // L, vec, 0)
    pltpu.sync_copy(pos_v.at[pl.ds(0, nedge)],
                    dlocb_hbm.at[pl.ds(c * E + base, nedge)])
    if nedge == SEG:
      pltpu.sync_copy(w_v, deg_sp.at[pos_v], add=True)
    else:
      pltpu.sync_copy(w_v.at[pl.ds(0, nedge)],
                      deg_sp.at[pos_v.at[pl.ds(0, nedge)]], add=True)

  @pl.loop(0, NSEG)
  def seg_loop(sg):
    do_seg(s * ET + sg * SEG, SEG)

  do_seg(s * ET + NSEG * SEG, TAIL)

  plsc.subcore_barrier()
  pltpu.sync_copy(deg_sp.at[pl.ds(s * _OUT_TEC, _OUT_TEC)],
                  zbuf.at[pl.ds(0, _OUT_TEC)])
  pltpu.sync_copy(zbuf.at[pl.ds(0, _OUT_TEC)],
                  out_hbm.at[pl.ds(c * NR + s * _OUT_TEC, _OUT_TEC)])


def _agg_body(dlocb_hbm, src_hbm, w_hbm, g_hbm, out_hbm, pos_a, pos_b, src_a,
              src_b, w_a, w_b, rows_a, rows_b, acc_sp, sem_a, sem_b):
  c = lax.axis_index("c")
  s = lax.axis_index("s")
  ebase = s * ET

  def zrows(i, _):
    rows_a[i, pl.ds(0, L)] = jnp.zeros((L,), jnp.float32)
    return 0

  lax.fori_loop(0, 128, zrows, 0)
  for j in range(_PER_TEC // 128):
    pltpu.sync_copy(rows_a.at[pl.ds(0, 128)],
                    acc_sp.at[pl.ds(s * _PER_TEC + j * 128, 128)])
  rem = _PER_TEC % 128
  if rem:
    pltpu.sync_copy(rows_a.at[pl.ds(0, rem)],
                    acc_sp.at[pl.ds(s * _PER_TEC + (_PER_TEC // 128) * 128,
                                    rem)])
  plsc.subcore_barrier()

  def load_meta(sg, pos_v, src_v, w_v):
    base = ebase + sg * SEG
    pltpu.sync_copy(dlocb_hbm.at[pl.ds(c * E + base, SEG)], pos_v)
    pltpu.sync_copy(src_hbm.at[pl.ds(base, SEG)], src_v)
    pltpu.sync_copy(w_hbm.at[pl.ds(base, SEG)], w_v)

  def scale(rows, w_v, nedge):
    def grp(g, _):
      wvec = w_v[pl.ds(g * L, L)]
      for i in range(L):
        k = g * L + i
        rows[k, pl.ds(0, L)] = rows[k, pl.ds(0, L)] * wvec[i]
      return 0

    lax.fori_loop(0, nedge // L, grp, 0)

  def process(rows, pos_v, w_v, sem):
    pltpu.make_async_copy(g_hbm.at[src_a], rows, sem).wait()
    scale(rows, w_v, SEG)
    pltpu.sync_copy(rows, acc_sp.at[pos_v], add=True)

  load_meta(0, pos_a, src_a, w_a)
  pltpu.async_copy(g_hbm.at[src_a], rows_a, sem_a)

  @pl.loop(0, NSEG, step=2)
  def seg_loop(sg):
    load_meta(sg + 1, pos_b, src_b, w_b)
    pltpu.async_copy(g_hbm.at[src_b], rows_b, sem_b)
    process(rows_a, pos_a, w_a, sem_a)

    @pl.when(sg + 2 < NSEG)
    def _():
      load_meta(sg + 2, pos_a, src_a, w_a)
      pltpu.async_copy(g_hbm.at[src_a], rows_a, sem_a)

    process(rows_b, pos_b, w_b, sem_b)

  tb = ebase + NSEG * SEG
  pltpu.sync_copy(dlocb_hbm.at[pl.ds(c * E + tb, TAIL)],
                  pos_a.at[pl.ds(0, TAIL)])
  pltpu.sync_copy(src_hbm.at[pl.ds(tb, TAIL)], src_a.at[pl.ds(0, TAIL)])
  pltpu.sync_copy(w_hbm.at[pl.ds(tb, TAIL)], w_a.at[pl.ds(0, TAIL)])
  pltpu.async_copy(g_hbm.at[src_a.at[pl.ds(0, TAIL)]],
                   rows_a.at[pl.ds(0, TAIL)], sem_a).wait()
  scale(rows_a, w_a, TAIL)
  pltpu.sync_copy(rows_a.at[pl.ds(0, TAIL)],
                  acc_sp.at[pos_a.at[pl.ds(0, TAIL)]], add=True)

  plsc.subcore_barrier()
  nfull = _OUT_TEC // 128
  for j in range(nfull):
    pltpu.sync_copy(acc_sp.at[pl.ds(s * _OUT_TEC + j * 128, 128)],
                    rows_a.at[pl.ds(0, 128)])
    pltpu.sync_copy(rows_a.at[pl.ds(0, 128)],
                    out_hbm.at[pl.ds(c * NR + s * _OUT_TEC + j * 128, 128)])
  rem = _OUT_TEC - nfull * 128
  if rem:
    pltpu.sync_copy(acc_sp.at[pl.ds(s * _OUT_TEC + nfull * 128, rem)],
                    rows_a.at[pl.ds(0, rem)])
    pltpu.sync_copy(rows_a.at[pl.ds(0, rem)],
                    out_hbm.at[pl.ds(c * NR + s * _OUT_TEC + nfull * 128,
                                     rem)])


_agg16 = functools.partial(
    pl.kernel,
    out_type=jax.ShapeDtypeStruct((NP, 16), jnp.float32),
    mesh=_mesh,
    scratch_types=[
        pltpu.VMEM((SEG,), jnp.int32),
        pltpu.VMEM((SEG,), jnp.int32),
        pltpu.VMEM((SEG,), jnp.int32),
        pltpu.VMEM((SEG,), jnp.int32),
        pltpu.VMEM((SEG,), jnp.float32),
        pltpu.VMEM((SEG,), jnp.float32),
        pltpu.VMEM((SEG, 16), jnp.float32),
        pltpu.VMEM((SEG, 16), jnp.float32),
        pltpu.VMEM_SHARED((ACC, 16), jnp.float32),
        pltpu.SemaphoreType.DMA,
        pltpu.SemaphoreType.DMA,
    ],
    compiler_params=pltpu.CompilerParams(use_tc_tiling_on_sc=False),
)(_agg_body)


def _t1_body(deg_ref, x_ref, dis_ref, gx_ref):
  deg = deg_ref[...] + 1.0
  dis = jnp.where(deg > 0, lax.rsqrt(deg), 0.0)
  dis_ref[...] = dis
  gx_ref[...] = x_ref[...] * dis


def _t2_body(aggx_ref, gx_ref, dis_ref, w1t_ref, b1_ref, g1_ref):
  dis = dis_ref[...]
  t = (aggx_ref[...] + gx_ref[...]) * dis
  h = jnp.dot(t, w1t_ref[...], preferred_element_type=jnp.float32)
  g1_ref[...] = jnp.maximum(h + b1_ref[...], 0.0) * dis


def _t3_body(agg1_ref, g1_ref, dis_ref, w2t_ref, b2_ref, wfct_ref, bfc_ref,
             o_ref):
  t = (agg1_ref[...] + g1_ref[...]) * dis_ref[...]
  h2 = jnp.maximum(
      jnp.dot(t, w2t_ref[...], preferred_element_type=jnp.float32)
      + b2_ref[...], 0.0)
  o_ref[...] = (jnp.dot(h2, wfct_ref[...], preferred_element_type=jnp.float32)
                + bfc_ref[...])


def kernel(x, edge_index, edge_weight, W1, b1, W2, b2, Wfc, bfc):
  src = edge_index[0]
  dst = edge_index[1]
  w = edge_weight

  xpad = jnp.pad(x, ((0, NP - N), (0, 16 - NCLS)))

  deg, dlocb = _k1_deg(dst, w)
  deg = deg[:, None]

  rb1 = 6256
  dis, gx = pl.pallas_call(
      _t1_body,
      grid=(NP // rb1,),
      in_specs=[
          pl.BlockSpec((rb1, 1), lambda i: (i, 0)),
          pl.BlockSpec((rb1, 16), lambda i: (i, 0)),
      ],
      out_specs=[
          pl.BlockSpec((rb1, 1), lambda i: (i, 0)),
          pl.BlockSpec((rb1, 16), lambda i: (i, 0)),
      ],
      out_shape=[
          jax.ShapeDtypeStruct((NP, 1), jnp.float32),
          jax.ShapeDtypeStruct((NP, 16), jnp.float32),
      ],
  )(deg, xpad)

  aggx = _agg16(dlocb, src, w, gx)

  w1t = jnp.pad(W1.T, ((0, 16 - NCLS), (0, 0)))
  rb2 = 3128
  g1 = pl.pallas_call(
      _t2_body,
      grid=(NP // rb2,),
      in_specs=[
          pl.BlockSpec((rb2, 16), lambda i: (i, 0)),
          pl.BlockSpec((rb2, 16), lambda i: (i, 0)),
          pl.BlockSpec((rb2, 1), lambda i: (i, 0)),
          pl.BlockSpec((16, HID), lambda i: (0, 0)),
          pl.BlockSpec((1, HID), lambda i: (0, 0)),
      ],
      out_specs=pl.BlockSpec((rb2, HID), lambda i: (i, 0)),
      out_shape=jax.ShapeDtypeStruct((NP, HID), jnp.float32),
  )(aggx, gx, dis, w1t, b1[None, :])

  agg1 = jnp.concatenate(
      [_agg16(dlocb, src, w,
              lax.slice(g1, (0, r * 16), (NP, (r + 1) * 16)))
       for r in range(8)], axis=1)

  rb3 = 3128
  out = pl.pallas_call(
      _t3_body,
      grid=(NP // rb3,),
      in_specs=[
          pl.BlockSpec((rb3, HID), lambda i: (i, 0)),
          pl.BlockSpec((rb3, HID), lambda i: (i, 0)),
          pl.BlockSpec((rb3, 1), lambda i: (i, 0)),
          pl.BlockSpec((HID, HID), lambda i: (0, 0)),
          pl.BlockSpec((1, HID), lambda i: (0, 0)),
          pl.BlockSpec((HID, NCLS), lambda i: (0, 0)),
          pl.BlockSpec((1, NCLS), lambda i: (0, 0)),
      ],
      out_specs=pl.BlockSpec((rb3, NCLS), lambda i: (i, 0)),
      out_shape=jax.ShapeDtypeStruct((NP, NCLS), jnp.float32),
  )(agg1, g1, dis, W2.T, b2[None, :], Wfc.T, bfc[None, :])

  return out[:N]
